# Initial kernel scaffold; baseline (speedup 1.0000x reference)
import jax
import jax.numpy as jnp
from jax.experimental import pallas as pl


def kernel(x, pos, batch, params):
    raise NotImplementedError("write your pallas kernel here")


# speedup vs baseline: 10.4126x; 10.4126x over previous
import functools

import jax
import jax.numpy as jnp
from jax import lax
from jax.experimental import pallas as pl
from jax.experimental.pallas import tpu as pltpu
from jax.experimental.pallas import tpu_sc as plsc

N_POINTS = 8192
K = 16
RATIO = 0.25
EPS_LN = 1e-5

_SC_CORES = 2
_SC_SUBCORES = 16
_SC_WORKERS = _SC_CORES * _SC_SUBCORES


def _sc_gather_many(tables, idx):
    B = idx.shape[0]
    assert B % (8 * _SC_WORKERS) == 0, B
    b_per_w = B // _SC_WORKERS
    dims = [int(t.shape[1]) for t in tables]
    row_bytes = 4 * sum(dims)
    chunk = b_per_w
    while chunk % 8 == 0 and (chunk // 2) % 8 == 0 and chunk * row_bytes > 262144:
        chunk //= 2
    nch = b_per_w // chunk

    mesh = plsc.VectorSubcoreMesh(core_axis_name="c", subcore_axis_name="s")
    out_type = [jax.ShapeDtypeStruct((B, d), jnp.float32) for d in dims]
    scratch = (
        [pltpu.VMEM((chunk,), jnp.int32)]
        + [pltpu.VMEM((chunk, d), jnp.float32) for d in dims]
        + [pltpu.SemaphoreType.DMA]
    )
    nt = len(tables)

    @functools.partial(
        pl.kernel, out_type=out_type, mesh=mesh, scratch_types=scratch,
        compiler_params=pltpu.CompilerParams(use_tc_tiling_on_sc=False))
    def gather_kernel(*refs):
        tbls = refs[:nt]
        idx_hbm = refs[nt]
        outs = refs[nt + 1:2 * nt + 1]
        idx_v = refs[2 * nt + 1]
        rows = refs[2 * nt + 2:3 * nt + 2]
        sem = refs[3 * nt + 2]
        wid = lax.axis_index("s") * _SC_CORES + lax.axis_index("c")
        base = wid * b_per_w

        def body(gi, carry):
            off = base + gi * chunk
            pltpu.sync_copy(idx_hbm.at[pl.ds(off, chunk)], idx_v)
            for t in range(nt):
                pltpu.async_copy(tbls[t].at[idx_v], rows[t], sem).wait()
                pltpu.sync_copy(rows[t], outs[t].at[pl.ds(off, chunk)])
            return carry

        lax.fori_loop(0, nch, body, 0)

    return gather_kernel(*tables, idx)


def _knn_kernel(a_ref, bt_ref, oi_ref, ov_ref, *, k, nb, rblk, excl_diag):
    pid = pl.program_id(0)
    a = a_ref[...]
    bt = bt_ref[...]
    aa = jnp.sum(a * a, axis=1, keepdims=True)
    bb = jnp.sum(bt * bt, axis=0, keepdims=True)
    ab = _mm(a, bt)
    d = (aa + bb) - 2.0 * ab
    col = lax.broadcasted_iota(jnp.int32, (rblk, nb), 1)
    if excl_diag:
        row = lax.broadcasted_iota(jnp.int32, (rblk, nb), 0) + pid * rblk
        d = jnp.where(col == row, jnp.inf, d)
    for t in range(k):
        mval = jnp.min(d, axis=1, keepdims=True)
        cand = jnp.where(d == mval, col, nb)
        midx = jnp.min(cand, axis=1)
        oi_ref[:, t] = midx
        ov_ref[:, t] = mval[:, 0]
        d = jnp.where(col == midx[:, None], jnp.inf, d)


def _knn_topk(a16, bt16, k, excl_diag):
    na = a16.shape[0]
    nb = bt16.shape[1]
    rblk = min(256, na)
    grid = (na // rblk,)
    kern = functools.partial(_knn_kernel, k=k, nb=nb, rblk=rblk,
                             excl_diag=excl_diag)
    return pl.pallas_call(
        kern,
        grid=grid,
        in_specs=[
            pl.BlockSpec((rblk, 16), lambda i: (i, 0)),
            pl.BlockSpec((16, nb), lambda i: (0, 0)),
        ],
        out_specs=[
            pl.BlockSpec((rblk, k), lambda i: (i, 0)),
            pl.BlockSpec((rblk, k), lambda i: (i, 0)),
        ],
        out_shape=[
            jax.ShapeDtypeStruct((na, k), jnp.int32),
            jax.ShapeDtypeStruct((na, k), jnp.float32),
        ],
    )(a16, bt16)


def _fps_kernel(px_ref, py_ref, pz_ref, o_ref, *, n, m, nrows):
    px = px_ref[...]
    py = py_ref[...]
    pz = pz_ref[...]
    qx0 = px[0, 0]
    qy0 = py[0, 0]
    qz0 = pz[0, 0]
    dx = px - qx0
    dy = py - qy0
    dz = pz - qz0
    d0 = (dx * dx + dy * dy) + dz * dz
    o_ref[0] = qx0
    o_ref[m] = qy0
    o_ref[2 * m] = qz0
    ii = (lax.broadcasted_iota(jnp.int32, (nrows, 128), 0) * 128
          + lax.broadcasted_iota(jnp.int32, (nrows, 128), 1))

    def body(i, d):
        mval = jnp.max(d)
        nxt = jnp.min(jnp.where(d == mval, ii, n))
        sel = ii == nxt
        qx = jnp.sum(jnp.where(sel, px, 0.0))
        qy = jnp.sum(jnp.where(sel, py, 0.0))
        qz = jnp.sum(jnp.where(sel, pz, 0.0))
        o_ref[i] = qx
        o_ref[m + i] = qy
        o_ref[2 * m + i] = qz
        ex = px - qx
        ey = py - qy
        ez = pz - qz
        dn = (ex * ex + ey * ey) + ez * ez
        return jnp.minimum(d, dn)

    lax.fori_loop(1, m, body, d0)


def _fps(pos, ratio):
    n = pos.shape[0]
    m = int(n * ratio)
    nrows = n // 128
    cols = [pos[:, i].reshape(nrows, 128) for i in range(3)]
    kern = functools.partial(_fps_kernel, n=n, m=m, nrows=nrows)
    flat = pl.pallas_call(
        kern,
        in_specs=[pl.BlockSpec((nrows, 128), lambda: (0, 0))] * 3,
        out_specs=pl.BlockSpec(memory_space=pltpu.SMEM),
        out_shape=jax.ShapeDtypeStruct((3 * m,), jnp.float32),
    )(*cols)
    return flat.reshape(3, m).T


def _ln(x, g, b):
    mu = jnp.mean(x, axis=-1, keepdims=True)
    v = jnp.mean((x - mu) ** 2, axis=-1, keepdims=True)
    return (x - mu) / jnp.sqrt(v + EPS_LN) * g + b


def _mm(x, wt):
    return jnp.dot(x.astype(jnp.bfloat16), wt.astype(jnp.bfloat16),
                   preferred_element_type=jnp.float32)


def _pre_kernel(x_ref, wpre_ref, bpre_ref, win_ref, bin_ref, g_ref, b_ref,
                wsrc_ref, wdst_ref, wl_ref, adst_ref, asrc_ref, xl_ref,
                *, has_pre):
    x = x_ref[...]
    if has_pre:
        x = jnp.maximum(_mm(x, wpre_ref[...]) + bpre_ref[...], 0.0)
    x1 = jnp.maximum(_mm(x, win_ref[...]) + bin_ref[...], 0.0)
    xn = _ln(x1, g_ref[...], b_ref[...])
    adst_ref[...] = _mm(xn, wdst_ref[...])
    asrc_ref[...] = _mm(xn, wsrc_ref[...])
    xl_ref[...] = _mm(xn, wl_ref[...])


def _block_prologue(x, p, pre=None):
    n, cin = x.shape
    c = p["lin"]["W"].shape[0]
    rblk = min(256, n)
    if pre is None:
        wpre = jnp.zeros((cin, cin), jnp.float32)
        bpre = jnp.zeros((1, cin), jnp.float32)
    else:
        wpre = pre["W"].T
        bpre = pre["b"][None, :]
    args = [
        x, wpre, bpre,
        p["lin_in"]["W"].T, p["lin_in"]["b"][None, :],
        p["ln"]["g"][None, :], p["ln"]["b"][None, :],
        p["lin_src"]["W"].T, p["lin_dst"]["W"].T, p["lin"]["W"].T,
    ]
    full = lambda a: pl.BlockSpec(a.shape, lambda i: (0,) * a.ndim)
    in_specs = [pl.BlockSpec((rblk, cin), lambda i: (i, 0))]
    in_specs += [full(a) for a in args[1:]]
    kern = functools.partial(_pre_kernel, has_pre=pre is not None)
    return pl.pallas_call(
        kern,
        grid=(n // rblk,),
        in_specs=in_specs,
        out_specs=[pl.BlockSpec((rblk, c), lambda i: (i, 0))] * 3,
        out_shape=[jax.ShapeDtypeStruct((n, c), jnp.float32)] * 3,
    )(*args)


def _conv_kernel(gsrc_ref, gxl_ref, gpos_ref, adst_ref, asrc_ref, xls_ref,
                 pos_ref, p1_ref, p1b_ref, p2_ref, p2b_ref,
                 a1_ref, a1b_ref, a2_ref, a2b_ref, wo_ref, wob_ref,
                 o_ref, *, rblk, k, c):
    gsrc = gsrc_ref[...]
    gxl = gxl_ref[...]
    gpos = gpos_ref[...]
    adst = adst_ref[...]
    asrc_i = asrc_ref[...]
    xl_i = xls_ref[...]
    pos_i = pos_ref[...]
    p1 = p1_ref[...]
    p1b = p1b_ref[...]
    p2 = p2_ref[...]
    p2b = p2b_ref[...]
    a1 = a1_ref[...]
    a1b = a1b_ref[...]
    a2 = a2_ref[...]
    a2b = a2b_ref[...]

    def mlp(h, w1, b1, w2, b2):
        h = jnp.maximum(_mm(h, w1) + b1, 0.0)
        return jnp.maximum(_mm(h, w2) + b2, 0.0)

    pm = (pos_i[:, None, :] - gpos).reshape(rblk * k, 16)
    delta = mlp(pm, p1, p1b, p2, p2b).reshape(rblk, k, c)
    alpha = adst[:, None, :] - gsrc + delta
    alpha = mlp(alpha.reshape(rblk * k, c), a1, a1b, a2, a2b)
    alpha = alpha.reshape(rblk, k, c)
    hs = jnp.maximum(p1b, 0.0)
    ds = jnp.maximum(_mm(hs, p2) + p2b, 0.0)
    alpha_s = mlp(adst - asrc_i + ds, a1, a1b, a2, a2b)

    amax = jnp.maximum(jnp.max(alpha, axis=1), alpha_s)
    e = jnp.exp(alpha - amax[:, None, :])
    es = jnp.exp(alpha_s - amax)
    denom = jnp.sum(e, axis=1) + es
    num = jnp.sum(e * (gxl + delta), axis=1) + es * (xl_i + ds)
    conv = num / (denom + 1e-16)
    o_ref[...] = jnp.maximum(_mm(conv, wo_ref[...]) + wob_ref[...], 0.0)


def _transformer_block(p, x, pos16, idx, pre=None):
    n = x.shape[0]
    c = p["lin"]["W"].shape[0]
    adst, asrc, xl = _block_prologue(x, p, pre=pre)
    gsrc, gxl, gpos = _sc_gather_many([asrc, xl, pos16], idx.reshape(-1))
    gsrc = gsrc.reshape(n, K, c)
    gxl = gxl.reshape(n, K, c)
    gpos = gpos.reshape(n, K, 16)

    rblk = max(32, min(256, (256 * 64) // c))
    rblk = min(rblk, n)
    p1 = jnp.zeros((16, 64), jnp.float32).at[:3].set(p["pos_nn"]["l1"]["W"].T)
    args = [
        gsrc, gxl, gpos, adst, asrc, xl, pos16,
        p1, p["pos_nn"]["l1"]["b"][None, :],
        p["pos_nn"]["l2"]["W"].T, p["pos_nn"]["l2"]["b"][None, :],
        p["attn_nn"]["l1"]["W"].T, p["attn_nn"]["l1"]["b"][None, :],
        p["attn_nn"]["l2"]["W"].T, p["attn_nn"]["l2"]["b"][None, :],
        p["lin_out"]["W"].T, p["lin_out"]["b"][None, :],
    ]
    full = lambda a: pl.BlockSpec(a.shape, lambda i: (0,) * a.ndim)
    in_specs = [
        pl.BlockSpec((rblk, K, c), lambda i: (i, 0, 0)),
        pl.BlockSpec((rblk, K, c), lambda i: (i, 0, 0)),
        pl.BlockSpec((rblk, K, 16), lambda i: (i, 0, 0)),
        pl.BlockSpec((rblk, c), lambda i: (i, 0)),
        pl.BlockSpec((rblk, c), lambda i: (i, 0)),
        pl.BlockSpec((rblk, c), lambda i: (i, 0)),
        pl.BlockSpec((rblk, 16), lambda i: (i, 0)),
    ] + [full(a) for a in args[7:]]
    kern = functools.partial(_conv_kernel, rblk=rblk, k=K, c=c)
    return pl.pallas_call(
        kern,
        grid=(n // rblk,),
        in_specs=in_specs,
        out_specs=pl.BlockSpec((rblk, c), lambda i: (i, 0)),
        out_shape=jax.ShapeDtypeStruct((n, c), jnp.float32),
    )(*args)


def _lna_kernel(x_ref, w_ref, b_ref, g_ref, bb_ref, o_ref):
    y = _mm(x_ref[...], w_ref[...]) + b_ref[...]
    o_ref[...] = jnp.maximum(_ln(y, g_ref[...], bb_ref[...]), 0.0)


def _lin_norm_act(p, x):
    n, cin = x.shape
    c = p["lin"]["W"].shape[0]
    rblk = min(256, n)
    args = [x, p["lin"]["W"].T, p["lin"]["b"][None, :],
            p["ln"]["g"][None, :], p["ln"]["b"][None, :]]
    full = lambda a: pl.BlockSpec(a.shape, lambda i: (0,) * a.ndim)
    in_specs = [pl.BlockSpec((rblk, cin), lambda i: (i, 0))]
    in_specs += [full(a) for a in args[1:]]
    return pl.pallas_call(
        _lna_kernel,
        grid=(n // rblk,),
        in_specs=in_specs,
        out_specs=pl.BlockSpec((rblk, c), lambda i: (i, 0)),
        out_shape=jax.ShapeDtypeStruct((n, c), jnp.float32),
    )(*args)


def _kmax_kernel(g_ref, o_ref):
    o_ref[...] = jnp.max(g_ref[...], axis=1)


def _kmax(g, m, c):
    rblk = min(256, m)
    return pl.pallas_call(
        _kmax_kernel,
        grid=(m // rblk,),
        in_specs=[pl.BlockSpec((rblk, K, c), lambda i: (i, 0, 0))],
        out_specs=pl.BlockSpec((rblk, c), lambda i: (i, 0)),
        out_shape=jax.ShapeDtypeStruct((m, c), jnp.float32),
    )(g)


def _up_kernel(x_ref, d3_ref, g3_ref, w_ref, b_ref, g_ref, bb_ref, o_ref):
    d3 = d3_ref[...]
    w = 1.0 / jnp.maximum(d3, 1e-16)
    w = w / jnp.sum(w, axis=1, keepdims=True)
    g3 = g3_ref[...]
    xi = jnp.sum(g3 * w[:, :, None], axis=1)
    y = _mm(x_ref[...], w_ref[...]) + b_ref[...]
    o_ref[...] = jnp.maximum(_ln(y, g_ref[...], bb_ref[...]), 0.0) + xi


def _transition_up(p, x_skip, x_sub, pos16, pos_sub16):
    n = x_skip.shape[0]
    cin = x_skip.shape[1]
    xs = _lin_norm_act(p["mlp_sub"], x_sub)
    idx3, d3 = _knn_topk(pos16, pos_sub16.T, 3, excl_diag=False)
    (g3,) = _sc_gather_many([xs], idx3.reshape(-1))
    g3 = g3.reshape(n, 3, cin)
    rblk = min(256, n)
    args = [x_skip, d3, g3, p["mlp"]["lin"]["W"].T,
            p["mlp"]["lin"]["b"][None, :],
            p["mlp"]["ln"]["g"][None, :], p["mlp"]["ln"]["b"][None, :]]
    full = lambda a: pl.BlockSpec(a.shape, lambda i: (0,) * a.ndim)
    in_specs = [
        pl.BlockSpec((rblk, cin), lambda i: (i, 0)),
        pl.BlockSpec((rblk, 3), lambda i: (i, 0)),
        pl.BlockSpec((rblk, 3, cin), lambda i: (i, 0, 0)),
    ] + [full(a) for a in args[3:]]
    return pl.pallas_call(
        _up_kernel,
        grid=(n // rblk,),
        in_specs=in_specs,
        out_specs=pl.BlockSpec((rblk, cin), lambda i: (i, 0)),
        out_shape=jax.ShapeDtypeStruct((n, cin), jnp.float32),
    )(*args)


def _pad16(pos):
    return jnp.pad(pos, ((0, 0), (0, 13)))


def kernel(x, pos, batch, params):
    del batch
    n0 = pos.shape[0]
    pos16 = [_pad16(pos)]
    idx = [_knn_topk(pos16[0], pos16[0].T, K, excl_diag=True)[0]]

    out_x = []
    x = _transformer_block(params["t_in"], x, pos16[0], idx[0])
    out_x.append(x)
    n = n0
    for i in range(2):
        sub_pos = _fps(pos16[i][:, :3], RATIO)
        m = sub_pos.shape[0]
        sp16 = _pad16(sub_pos)
        ai, _ = _knn_topk(sp16, pos16[i].T, K, excl_diag=False)
        xt = _lin_norm_act(params["td"][i]["mlp"], x)
        (g,) = _sc_gather_many([xt], ai.reshape(-1))
        c = xt.shape[1]
        x = _kmax(g.reshape(m, K, c), m, c)
        pos16.append(sp16)
        idx.append(_knn_topk(sp16, sp16.T, K, excl_diag=True)[0])
        x = _transformer_block(params["tfd"][i], x, sp16, idx[i + 1])
        out_x.append(x)
        n = m

    x = _transformer_block(params["t_sum"], x, pos16[2], idx[2],
                           pre=params["mlp_summit"])

    for i in range(2):
        lev = 1 - i
        x = _transition_up(params["tu"][lev], out_x[lev], x,
                           pos16[lev], pos16[lev + 1])
        x = _transformer_block(params["tfu"][lev], x, pos16[lev], idx[lev])
    return x


# speedup vs baseline: 10.4128x; 1.0000x over previous
import functools

import jax
import jax.numpy as jnp
from jax import lax
from jax.experimental import pallas as pl
from jax.experimental.pallas import tpu as pltpu
from jax.experimental.pallas import tpu_sc as plsc

N_POINTS = 8192
K = 16
RATIO = 0.25
EPS_LN = 1e-5

_SC_CORES = 2
_SC_SUBCORES = 16
_SC_WORKERS = _SC_CORES * _SC_SUBCORES


def _sc_gather_many(tables, idx):
    B = idx.shape[0]
    assert B % (8 * _SC_WORKERS) == 0, B
    b_per_w = B // _SC_WORKERS
    dims = [int(t.shape[1]) for t in tables]
    row_bytes = 4 * sum(dims)
    chunk = b_per_w
    while chunk % 8 == 0 and (chunk // 2) % 8 == 0 and chunk * row_bytes > 262144:
        chunk //= 2
    nch = b_per_w // chunk

    mesh = plsc.VectorSubcoreMesh(core_axis_name="c", subcore_axis_name="s")
    out_type = [jax.ShapeDtypeStruct((B, d), jnp.float32) for d in dims]
    scratch = (
        [pltpu.VMEM((chunk,), jnp.int32)]
        + [pltpu.VMEM((chunk, d), jnp.float32) for d in dims]
        + [pltpu.SemaphoreType.DMA]
    )
    nt = len(tables)

    @functools.partial(
        pl.kernel, out_type=out_type, mesh=mesh, scratch_types=scratch,
        compiler_params=pltpu.CompilerParams(use_tc_tiling_on_sc=False))
    def gather_kernel(*refs):
        tbls = refs[:nt]
        idx_hbm = refs[nt]
        outs = refs[nt + 1:2 * nt + 1]
        idx_v = refs[2 * nt + 1]
        rows = refs[2 * nt + 2:3 * nt + 2]
        sem = refs[3 * nt + 2]
        wid = lax.axis_index("s") * _SC_CORES + lax.axis_index("c")
        base = wid * b_per_w

        def body(gi, carry):
            off = base + gi * chunk
            pltpu.sync_copy(idx_hbm.at[pl.ds(off, chunk)], idx_v)
            for t in range(nt):
                pltpu.async_copy(tbls[t].at[idx_v], rows[t], sem).wait()
                pltpu.sync_copy(rows[t], outs[t].at[pl.ds(off, chunk)])
            return carry

        lax.fori_loop(0, nch, body, 0)

    return gather_kernel(*tables, idx)


def _knn_kernel(a_ref, bt_ref, oi_ref, ov_ref, *, k, nb, rblk, excl_diag):
    pid = pl.program_id(0)
    a = a_ref[...]
    bt = bt_ref[...]
    aa = jnp.sum(a * a, axis=1, keepdims=True)
    bb = jnp.sum(bt * bt, axis=0, keepdims=True)
    ab = _mm(a, bt)
    d = (aa + bb) - 2.0 * ab
    col = lax.broadcasted_iota(jnp.int32, (rblk, nb), 1)
    if excl_diag:
        row = lax.broadcasted_iota(jnp.int32, (rblk, nb), 0) + pid * rblk
        d = jnp.where(col == row, jnp.inf, d)
    for t in range(k):
        mval = jnp.min(d, axis=1, keepdims=True)
        cand = jnp.where(d == mval, col, nb)
        midx = jnp.min(cand, axis=1)
        oi_ref[:, t] = midx
        ov_ref[:, t] = mval[:, 0]
        d = jnp.where(col == midx[:, None], jnp.inf, d)


def _knn_topk(a16, bt16, k, excl_diag):
    na = a16.shape[0]
    nb = bt16.shape[1]
    rblk = min(256, na)
    grid = (na // rblk,)
    kern = functools.partial(_knn_kernel, k=k, nb=nb, rblk=rblk,
                             excl_diag=excl_diag)
    return pl.pallas_call(
        kern,
        grid=grid,
        in_specs=[
            pl.BlockSpec((rblk, 16), lambda i: (i, 0)),
            pl.BlockSpec((16, nb), lambda i: (0, 0)),
        ],
        out_specs=[
            pl.BlockSpec((rblk, k), lambda i: (i, 0)),
            pl.BlockSpec((rblk, k), lambda i: (i, 0)),
        ],
        out_shape=[
            jax.ShapeDtypeStruct((na, k), jnp.int32),
            jax.ShapeDtypeStruct((na, k), jnp.float32),
        ],
    )(a16, bt16)


def _fps_kernel(px_ref, py_ref, pz_ref, o_ref, *, n, m, nrows):
    px = px_ref[...]
    py = py_ref[...]
    pz = pz_ref[...]
    qx0 = px[0, 0]
    qy0 = py[0, 0]
    qz0 = pz[0, 0]
    dx = px - qx0
    dy = py - qy0
    dz = pz - qz0
    d0 = (dx * dx + dy * dy) + dz * dz
    o_ref[0] = qx0
    o_ref[m] = qy0
    o_ref[2 * m] = qz0
    ii = (lax.broadcasted_iota(jnp.int32, (nrows, 128), 0) * 128
          + lax.broadcasted_iota(jnp.int32, (nrows, 128), 1))

    lane = lax.broadcasted_iota(jnp.int32, (1, 128), 1)

    def body(i, d):
        mval = jnp.max(d)
        nxt = jnp.min(jnp.where(d == mval, ii, n))
        r = nxt // 128
        csel = lane == (nxt - r * 128)
        qx = jnp.sum(jnp.where(csel, px_ref[pl.ds(r, 1), :], 0.0))
        qy = jnp.sum(jnp.where(csel, py_ref[pl.ds(r, 1), :], 0.0))
        qz = jnp.sum(jnp.where(csel, pz_ref[pl.ds(r, 1), :], 0.0))
        o_ref[i] = qx
        o_ref[m + i] = qy
        o_ref[2 * m + i] = qz
        ex = px - qx
        ey = py - qy
        ez = pz - qz
        dn = (ex * ex + ey * ey) + ez * ez
        return jnp.minimum(d, dn)

    lax.fori_loop(1, m, body, d0)


def _fps(pos, ratio):
    n = pos.shape[0]
    m = int(n * ratio)
    nrows = n // 128
    cols = [pos[:, i].reshape(nrows, 128) for i in range(3)]
    kern = functools.partial(_fps_kernel, n=n, m=m, nrows=nrows)
    flat = pl.pallas_call(
        kern,
        in_specs=[pl.BlockSpec((nrows, 128), lambda: (0, 0))] * 3,
        out_specs=pl.BlockSpec(memory_space=pltpu.SMEM),
        out_shape=jax.ShapeDtypeStruct((3 * m,), jnp.float32),
    )(*cols)
    return flat.reshape(3, m).T


def _ln(x, g, b):
    mu = jnp.mean(x, axis=-1, keepdims=True)
    v = jnp.mean((x - mu) ** 2, axis=-1, keepdims=True)
    return (x - mu) / jnp.sqrt(v + EPS_LN) * g + b


def _mm(x, wt):
    return jnp.dot(x.astype(jnp.bfloat16), wt.astype(jnp.bfloat16),
                   preferred_element_type=jnp.float32)


def _pre_kernel(x_ref, wpre_ref, bpre_ref, win_ref, bin_ref, g_ref, b_ref,
                wsrc_ref, wdst_ref, wl_ref, adst_ref, asrc_ref, xl_ref,
                *, has_pre):
    x = x_ref[...]
    if has_pre:
        x = jnp.maximum(_mm(x, wpre_ref[...]) + bpre_ref[...], 0.0)
    x1 = jnp.maximum(_mm(x, win_ref[...]) + bin_ref[...], 0.0)
    xn = _ln(x1, g_ref[...], b_ref[...])
    adst_ref[...] = _mm(xn, wdst_ref[...])
    asrc_ref[...] = _mm(xn, wsrc_ref[...])
    xl_ref[...] = _mm(xn, wl_ref[...])


def _block_prologue(x, p, pre=None):
    n, cin = x.shape
    c = p["lin"]["W"].shape[0]
    rblk = min(256, n)
    if pre is None:
        wpre = jnp.zeros((cin, cin), jnp.float32)
        bpre = jnp.zeros((1, cin), jnp.float32)
    else:
        wpre = pre["W"].T
        bpre = pre["b"][None, :]
    args = [
        x, wpre, bpre,
        p["lin_in"]["W"].T, p["lin_in"]["b"][None, :],
        p["ln"]["g"][None, :], p["ln"]["b"][None, :],
        p["lin_src"]["W"].T, p["lin_dst"]["W"].T, p["lin"]["W"].T,
    ]
    full = lambda a: pl.BlockSpec(a.shape, lambda i: (0,) * a.ndim)
    in_specs = [pl.BlockSpec((rblk, cin), lambda i: (i, 0))]
    in_specs += [full(a) for a in args[1:]]
    kern = functools.partial(_pre_kernel, has_pre=pre is not None)
    return pl.pallas_call(
        kern,
        grid=(n // rblk,),
        in_specs=in_specs,
        out_specs=[pl.BlockSpec((rblk, c), lambda i: (i, 0))] * 3,
        out_shape=[jax.ShapeDtypeStruct((n, c), jnp.float32)] * 3,
    )(*args)


def _conv_kernel(gsrc_ref, gxl_ref, gpos_ref, adst_ref, asrc_ref, xls_ref,
                 pos_ref, p1_ref, p1b_ref, p2_ref, p2b_ref,
                 a1_ref, a1b_ref, a2_ref, a2b_ref, wo_ref, wob_ref,
                 o_ref, *, rblk, k, c):
    gsrc = gsrc_ref[...]
    gxl = gxl_ref[...]
    gpos = gpos_ref[...]
    adst = adst_ref[...]
    asrc_i = asrc_ref[...]
    xl_i = xls_ref[...]
    pos_i = pos_ref[...]
    p1 = p1_ref[...]
    p1b = p1b_ref[...]
    p2 = p2_ref[...]
    p2b = p2b_ref[...]
    a1 = a1_ref[...]
    a1b = a1b_ref[...]
    a2 = a2_ref[...]
    a2b = a2b_ref[...]

    def mlp(h, w1, b1, w2, b2):
        h = jnp.maximum(_mm(h, w1) + b1, 0.0)
        return jnp.maximum(_mm(h, w2) + b2, 0.0)

    pm = (pos_i[:, None, :] - gpos).reshape(rblk * k, 16)
    delta = mlp(pm, p1, p1b, p2, p2b).reshape(rblk, k, c)
    alpha = adst[:, None, :] - gsrc + delta
    alpha = mlp(alpha.reshape(rblk * k, c), a1, a1b, a2, a2b)
    alpha = alpha.reshape(rblk, k, c)
    hs = jnp.maximum(p1b, 0.0)
    ds = jnp.maximum(_mm(hs, p2) + p2b, 0.0)
    alpha_s = mlp(adst - asrc_i + ds, a1, a1b, a2, a2b)

    amax = jnp.maximum(jnp.max(alpha, axis=1), alpha_s)
    e = jnp.exp(alpha - amax[:, None, :])
    es = jnp.exp(alpha_s - amax)
    denom = jnp.sum(e, axis=1) + es
    num = jnp.sum(e * (gxl + delta), axis=1) + es * (xl_i + ds)
    conv = num / (denom + 1e-16)
    o_ref[...] = jnp.maximum(_mm(conv, wo_ref[...]) + wob_ref[...], 0.0)


def _transformer_block(p, x, pos16, idx, pre=None):
    n = x.shape[0]
    c = p["lin"]["W"].shape[0]
    adst, asrc, xl = _block_prologue(x, p, pre=pre)
    gsrc, gxl, gpos = _sc_gather_many([asrc, xl, pos16], idx.reshape(-1))
    gsrc = gsrc.reshape(n, K, c)
    gxl = gxl.reshape(n, K, c)
    gpos = gpos.reshape(n, K, 16)

    rblk = max(32, min(256, (256 * 64) // c))
    rblk = min(rblk, n)
    p1 = jnp.zeros((16, 64), jnp.float32).at[:3].set(p["pos_nn"]["l1"]["W"].T)
    args = [
        gsrc, gxl, gpos, adst, asrc, xl, pos16,
        p1, p["pos_nn"]["l1"]["b"][None, :],
        p["pos_nn"]["l2"]["W"].T, p["pos_nn"]["l2"]["b"][None, :],
        p["attn_nn"]["l1"]["W"].T, p["attn_nn"]["l1"]["b"][None, :],
        p["attn_nn"]["l2"]["W"].T, p["attn_nn"]["l2"]["b"][None, :],
        p["lin_out"]["W"].T, p["lin_out"]["b"][None, :],
    ]
    full = lambda a: pl.BlockSpec(a.shape, lambda i: (0,) * a.ndim)
    in_specs = [
        pl.BlockSpec((rblk, K, c), lambda i: (i, 0, 0)),
        pl.BlockSpec((rblk, K, c), lambda i: (i, 0, 0)),
        pl.BlockSpec((rblk, K, 16), lambda i: (i, 0, 0)),
        pl.BlockSpec((rblk, c), lambda i: (i, 0)),
        pl.BlockSpec((rblk, c), lambda i: (i, 0)),
        pl.BlockSpec((rblk, c), lambda i: (i, 0)),
        pl.BlockSpec((rblk, 16), lambda i: (i, 0)),
    ] + [full(a) for a in args[7:]]
    kern = functools.partial(_conv_kernel, rblk=rblk, k=K, c=c)
    return pl.pallas_call(
        kern,
        grid=(n // rblk,),
        in_specs=in_specs,
        out_specs=pl.BlockSpec((rblk, c), lambda i: (i, 0)),
        out_shape=jax.ShapeDtypeStruct((n, c), jnp.float32),
    )(*args)


def _lna_kernel(x_ref, w_ref, b_ref, g_ref, bb_ref, o_ref):
    y = _mm(x_ref[...], w_ref[...]) + b_ref[...]
    o_ref[...] = jnp.maximum(_ln(y, g_ref[...], bb_ref[...]), 0.0)


def _lin_norm_act(p, x):
    n, cin = x.shape
    c = p["lin"]["W"].shape[0]
    rblk = min(256, n)
    args = [x, p["lin"]["W"].T, p["lin"]["b"][None, :],
            p["ln"]["g"][None, :], p["ln"]["b"][None, :]]
    full = lambda a: pl.BlockSpec(a.shape, lambda i: (0,) * a.ndim)
    in_specs = [pl.BlockSpec((rblk, cin), lambda i: (i, 0))]
    in_specs += [full(a) for a in args[1:]]
    return pl.pallas_call(
        _lna_kernel,
        grid=(n // rblk,),
        in_specs=in_specs,
        out_specs=pl.BlockSpec((rblk, c), lambda i: (i, 0)),
        out_shape=jax.ShapeDtypeStruct((n, c), jnp.float32),
    )(*args)


def _kmax_kernel(g_ref, o_ref):
    o_ref[...] = jnp.max(g_ref[...], axis=1)


def _kmax(g, m, c):
    rblk = min(256, m)
    return pl.pallas_call(
        _kmax_kernel,
        grid=(m // rblk,),
        in_specs=[pl.BlockSpec((rblk, K, c), lambda i: (i, 0, 0))],
        out_specs=pl.BlockSpec((rblk, c), lambda i: (i, 0)),
        out_shape=jax.ShapeDtypeStruct((m, c), jnp.float32),
    )(g)


def _up_kernel(x_ref, d3_ref, g3_ref, w_ref, b_ref, g_ref, bb_ref, o_ref):
    d3 = d3_ref[...]
    w = 1.0 / jnp.maximum(d3, 1e-16)
    w = w / jnp.sum(w, axis=1, keepdims=True)
    g3 = g3_ref[...]
    xi = jnp.sum(g3 * w[:, :, None], axis=1)
    y = _mm(x_ref[...], w_ref[...]) + b_ref[...]
    o_ref[...] = jnp.maximum(_ln(y, g_ref[...], bb_ref[...]), 0.0) + xi


def _transition_up(p, x_skip, x_sub, pos16, pos_sub16):
    n = x_skip.shape[0]
    cin = x_skip.shape[1]
    xs = _lin_norm_act(p["mlp_sub"], x_sub)
    idx3, d3 = _knn_topk(pos16, pos_sub16.T, 3, excl_diag=False)
    (g3,) = _sc_gather_many([xs], idx3.reshape(-1))
    g3 = g3.reshape(n, 3, cin)
    rblk = min(256, n)
    args = [x_skip, d3, g3, p["mlp"]["lin"]["W"].T,
            p["mlp"]["lin"]["b"][None, :],
            p["mlp"]["ln"]["g"][None, :], p["mlp"]["ln"]["b"][None, :]]
    full = lambda a: pl.BlockSpec(a.shape, lambda i: (0,) * a.ndim)
    in_specs = [
        pl.BlockSpec((rblk, cin), lambda i: (i, 0)),
        pl.BlockSpec((rblk, 3), lambda i: (i, 0)),
        pl.BlockSpec((rblk, 3, cin), lambda i: (i, 0, 0)),
    ] + [full(a) for a in args[3:]]
    return pl.pallas_call(
        _up_kernel,
        grid=(n // rblk,),
        in_specs=in_specs,
        out_specs=pl.BlockSpec((rblk, cin), lambda i: (i, 0)),
        out_shape=jax.ShapeDtypeStruct((n, cin), jnp.float32),
    )(*args)


def _pad16(pos):
    return jnp.pad(pos, ((0, 0), (0, 13)))


def kernel(x, pos, batch, params):
    del batch
    n0 = pos.shape[0]
    pos16 = [_pad16(pos)]
    idx = [_knn_topk(pos16[0], pos16[0].T, K, excl_diag=True)[0]]

    out_x = []
    x = _transformer_block(params["t_in"], x, pos16[0], idx[0])
    out_x.append(x)
    n = n0
    for i in range(2):
        sub_pos = _fps(pos16[i][:, :3], RATIO)
        m = sub_pos.shape[0]
        sp16 = _pad16(sub_pos)
        ai, _ = _knn_topk(sp16, pos16[i].T, K, excl_diag=False)
        xt = _lin_norm_act(params["td"][i]["mlp"], x)
        (g,) = _sc_gather_many([xt], ai.reshape(-1))
        c = xt.shape[1]
        x = _kmax(g.reshape(m, K, c), m, c)
        pos16.append(sp16)
        idx.append(_knn_topk(sp16, sp16.T, K, excl_diag=True)[0])
        x = _transformer_block(params["tfd"][i], x, sp16, idx[i + 1])
        out_x.append(x)
        n = m

    x = _transformer_block(params["t_sum"], x, pos16[2], idx[2],
                           pre=params["mlp_summit"])

    for i in range(2):
        lev = 1 - i
        x = _transition_up(params["tu"][lev], out_x[lev], x,
                           pos16[lev], pos16[lev + 1])
        x = _transformer_block(params["tfu"][lev], x, pos16[lev], idx[lev])
    return x


# speedup vs baseline: 10.9816x; 1.0546x over previous
import functools

import jax
import jax.numpy as jnp
from jax import lax
from jax.experimental import pallas as pl
from jax.experimental.pallas import tpu as pltpu
from jax.experimental.pallas import tpu_sc as plsc

N_POINTS = 8192
K = 16
RATIO = 0.25
EPS_LN = 1e-5

_SC_CORES = 2
_SC_SUBCORES = 16
_SC_WORKERS = _SC_CORES * _SC_SUBCORES


def _sc_gather_many(tables, idx):
    B = idx.shape[0]
    assert B % (8 * _SC_WORKERS) == 0, B
    b_per_w = B // _SC_WORKERS
    dims = [int(t.shape[1]) for t in tables]
    row_bytes = 4 * sum(dims)
    chunk = b_per_w
    while chunk % 8 == 0 and (chunk // 2) % 8 == 0 and chunk * row_bytes > 262144:
        chunk //= 2
    nch = b_per_w // chunk

    mesh = plsc.VectorSubcoreMesh(core_axis_name="c", subcore_axis_name="s")
    out_type = [jax.ShapeDtypeStruct((B, d), jnp.float32) for d in dims]
    scratch = (
        [pltpu.VMEM((chunk,), jnp.int32)]
        + [pltpu.VMEM((chunk, d), jnp.float32) for d in dims]
        + [pltpu.SemaphoreType.DMA]
    )
    nt = len(tables)

    @functools.partial(
        pl.kernel, out_type=out_type, mesh=mesh, scratch_types=scratch,
        compiler_params=pltpu.CompilerParams(use_tc_tiling_on_sc=False))
    def gather_kernel(*refs):
        tbls = refs[:nt]
        idx_hbm = refs[nt]
        outs = refs[nt + 1:2 * nt + 1]
        idx_v = refs[2 * nt + 1]
        rows = refs[2 * nt + 2:3 * nt + 2]
        sem = refs[3 * nt + 2]
        wid = lax.axis_index("s") * _SC_CORES + lax.axis_index("c")
        base = wid * b_per_w

        def body(gi, carry):
            off = base + gi * chunk
            pltpu.sync_copy(idx_hbm.at[pl.ds(off, chunk)], idx_v)
            for t in range(nt):
                pltpu.async_copy(tbls[t].at[idx_v], rows[t], sem).wait()
                pltpu.sync_copy(rows[t], outs[t].at[pl.ds(off, chunk)])
            return carry

        lax.fori_loop(0, nch, body, 0)

    return gather_kernel(*tables, idx)


def _knn_kernel(a_ref, bt_ref, oi_ref, ov_ref, *, k, nb, rblk, excl_diag):
    pid = pl.program_id(0)
    a = a_ref[...]
    bt = bt_ref[...]
    aa = jnp.sum(a * a, axis=1, keepdims=True)
    bb = jnp.sum(bt * bt, axis=0, keepdims=True)
    ab = _mm(a, bt)
    d = (aa + bb) - 2.0 * ab
    col = lax.broadcasted_iota(jnp.int32, (rblk, nb), 1)
    if excl_diag:
        row = lax.broadcasted_iota(jnp.int32, (rblk, nb), 0) + pid * rblk
        d = jnp.where(col == row, jnp.inf, d)
    for t in range(k):
        mval = jnp.min(d, axis=1, keepdims=True)
        cand = jnp.where(d == mval, col, nb)
        midx = jnp.min(cand, axis=1)
        oi_ref[:, t] = midx
        ov_ref[:, t] = mval[:, 0]
        d = jnp.where(col == midx[:, None], jnp.inf, d)


def _knn_topk(a16, bt16, k, excl_diag):
    na = a16.shape[0]
    nb = bt16.shape[1]
    rblk = min(256, na)
    grid = (na // rblk,)
    kern = functools.partial(_knn_kernel, k=k, nb=nb, rblk=rblk,
                             excl_diag=excl_diag)
    return pl.pallas_call(
        kern,
        grid=grid,
        in_specs=[
            pl.BlockSpec((rblk, 16), lambda i: (i, 0)),
            pl.BlockSpec((16, nb), lambda i: (0, 0)),
        ],
        out_specs=[
            pl.BlockSpec((rblk, k), lambda i: (i, 0)),
            pl.BlockSpec((rblk, k), lambda i: (i, 0)),
        ],
        out_shape=[
            jax.ShapeDtypeStruct((na, k), jnp.int32),
            jax.ShapeDtypeStruct((na, k), jnp.float32),
        ],
    )(a16, bt16)


def _fps_kernel(px_ref, py_ref, pz_ref, o_ref, *, n, m, nrows):
    px = px_ref[...]
    py = py_ref[...]
    pz = pz_ref[...]
    qx0 = px[0, 0]
    qy0 = py[0, 0]
    qz0 = pz[0, 0]
    dx = px - qx0
    dy = py - qy0
    dz = pz - qz0
    d0 = (dx * dx + dy * dy) + dz * dz
    o_ref[0] = qx0
    o_ref[m] = qy0
    o_ref[2 * m] = qz0
    ii = (lax.broadcasted_iota(jnp.int32, (nrows, 128), 0) * 128
          + lax.broadcasted_iota(jnp.int32, (nrows, 128), 1))

    def red2(x, op):
        return op(op(x, axis=0, keepdims=True), axis=1, keepdims=True)

    def body(i, d):
        mval = red2(d, jnp.max)
        sel = d == mval
        nxt = red2(jnp.where(sel, ii, n), jnp.min)
        psel = ii == nxt
        qx = red2(jnp.where(psel, px, 0.0), jnp.sum)
        qy = red2(jnp.where(psel, py, 0.0), jnp.sum)
        qz = red2(jnp.where(psel, pz, 0.0), jnp.sum)
        o_ref[i] = qx[0, 0]
        o_ref[m + i] = qy[0, 0]
        o_ref[2 * m + i] = qz[0, 0]
        ex = px - qx
        ey = py - qy
        ez = pz - qz
        dn = (ex * ex + ey * ey) + ez * ez
        return jnp.minimum(d, dn)

    lax.fori_loop(1, m, body, d0)


def _fps(pos, ratio):
    n = pos.shape[0]
    m = int(n * ratio)
    nrows = n // 128
    cols = [pos[:, i].reshape(nrows, 128) for i in range(3)]
    kern = functools.partial(_fps_kernel, n=n, m=m, nrows=nrows)
    flat = pl.pallas_call(
        kern,
        in_specs=[pl.BlockSpec((nrows, 128), lambda: (0, 0))] * 3,
        out_specs=pl.BlockSpec(memory_space=pltpu.SMEM),
        out_shape=jax.ShapeDtypeStruct((3 * m,), jnp.float32),
    )(*cols)
    return flat.reshape(3, m).T


def _ln(x, g, b):
    mu = jnp.mean(x, axis=-1, keepdims=True)
    v = jnp.mean((x - mu) ** 2, axis=-1, keepdims=True)
    return (x - mu) / jnp.sqrt(v + EPS_LN) * g + b


def _mm(x, wt):
    return jnp.dot(x.astype(jnp.bfloat16), wt.astype(jnp.bfloat16),
                   preferred_element_type=jnp.float32)


def _pre_kernel(x_ref, wpre_ref, bpre_ref, win_ref, bin_ref, g_ref, b_ref,
                wsrc_ref, wdst_ref, wl_ref, adst_ref, asrc_ref, xl_ref,
                *, has_pre):
    x = x_ref[...]
    if has_pre:
        x = jnp.maximum(_mm(x, wpre_ref[...]) + bpre_ref[...], 0.0)
    x1 = jnp.maximum(_mm(x, win_ref[...]) + bin_ref[...], 0.0)
    xn = _ln(x1, g_ref[...], b_ref[...])
    adst_ref[...] = _mm(xn, wdst_ref[...])
    asrc_ref[...] = _mm(xn, wsrc_ref[...])
    xl_ref[...] = _mm(xn, wl_ref[...])


def _block_prologue(x, p, pre=None):
    n, cin = x.shape
    c = p["lin"]["W"].shape[0]
    rblk = min(256, n)
    if pre is None:
        wpre = jnp.zeros((cin, cin), jnp.float32)
        bpre = jnp.zeros((1, cin), jnp.float32)
    else:
        wpre = pre["W"].T
        bpre = pre["b"][None, :]
    args = [
        x, wpre, bpre,
        p["lin_in"]["W"].T, p["lin_in"]["b"][None, :],
        p["ln"]["g"][None, :], p["ln"]["b"][None, :],
        p["lin_src"]["W"].T, p["lin_dst"]["W"].T, p["lin"]["W"].T,
    ]
    full = lambda a: pl.BlockSpec(a.shape, lambda i: (0,) * a.ndim)
    in_specs = [pl.BlockSpec((rblk, cin), lambda i: (i, 0))]
    in_specs += [full(a) for a in args[1:]]
    kern = functools.partial(_pre_kernel, has_pre=pre is not None)
    return pl.pallas_call(
        kern,
        grid=(n // rblk,),
        in_specs=in_specs,
        out_specs=[pl.BlockSpec((rblk, c), lambda i: (i, 0))] * 3,
        out_shape=[jax.ShapeDtypeStruct((n, c), jnp.float32)] * 3,
    )(*args)


def _conv_kernel(gsrc_ref, gxl_ref, gpos_ref, adst_ref, asrc_ref, xls_ref,
                 pos_ref, p1_ref, p1b_ref, p2_ref, p2b_ref,
                 a1_ref, a1b_ref, a2_ref, a2b_ref, wo_ref, wob_ref,
                 o_ref, *, rblk, k, c):
    gsrc = gsrc_ref[...].reshape(rblk, k, c)
    gxl = gxl_ref[...].reshape(rblk, k, c)
    gpos = gpos_ref[...].reshape(rblk, k, 16)
    adst = adst_ref[...]
    asrc_i = asrc_ref[...]
    xl_i = xls_ref[...]
    pos_i = pos_ref[...]
    p1 = p1_ref[...]
    p1b = p1b_ref[...]
    p2 = p2_ref[...]
    p2b = p2b_ref[...]
    a1 = a1_ref[...]
    a1b = a1b_ref[...]
    a2 = a2_ref[...]
    a2b = a2b_ref[...]

    def mlp(h, w1, b1, w2, b2):
        h = jnp.maximum(_mm(h, w1) + b1, 0.0)
        return jnp.maximum(_mm(h, w2) + b2, 0.0)

    pm = (pos_i[:, None, :] - gpos).reshape(rblk * k, 16)
    delta = mlp(pm, p1, p1b, p2, p2b).reshape(rblk, k, c)
    alpha = adst[:, None, :] - gsrc + delta
    alpha = mlp(alpha.reshape(rblk * k, c), a1, a1b, a2, a2b)
    alpha = alpha.reshape(rblk, k, c)
    hs = jnp.maximum(p1b, 0.0)
    ds = jnp.maximum(_mm(hs, p2) + p2b, 0.0)
    alpha_s = mlp(adst - asrc_i + ds, a1, a1b, a2, a2b)

    amax = jnp.maximum(jnp.max(alpha, axis=1), alpha_s)
    e = jnp.exp(alpha - amax[:, None, :])
    es = jnp.exp(alpha_s - amax)
    denom = jnp.sum(e, axis=1) + es
    num = jnp.sum(e * (gxl + delta), axis=1) + es * (xl_i + ds)
    conv = num / (denom + 1e-16)
    o_ref[...] = jnp.maximum(_mm(conv, wo_ref[...]) + wob_ref[...], 0.0)


def _transformer_block(p, x, pos16, idx, pre=None):
    n = x.shape[0]
    c = p["lin"]["W"].shape[0]
    adst, asrc, xl = _block_prologue(x, p, pre=pre)
    gsrc, gxl, gpos = _sc_gather_many([asrc, xl, pos16], idx.reshape(-1))

    rblk = max(32, min(256, (256 * 64) // c))
    rblk = min(rblk, n)
    p1 = jnp.zeros((16, 64), jnp.float32).at[:3].set(p["pos_nn"]["l1"]["W"].T)
    args = [
        gsrc, gxl, gpos, adst, asrc, xl, pos16,
        p1, p["pos_nn"]["l1"]["b"][None, :],
        p["pos_nn"]["l2"]["W"].T, p["pos_nn"]["l2"]["b"][None, :],
        p["attn_nn"]["l1"]["W"].T, p["attn_nn"]["l1"]["b"][None, :],
        p["attn_nn"]["l2"]["W"].T, p["attn_nn"]["l2"]["b"][None, :],
        p["lin_out"]["W"].T, p["lin_out"]["b"][None, :],
    ]
    full = lambda a: pl.BlockSpec(a.shape, lambda i: (0,) * a.ndim)
    in_specs = [
        pl.BlockSpec((rblk * K, c), lambda i: (i, 0)),
        pl.BlockSpec((rblk * K, c), lambda i: (i, 0)),
        pl.BlockSpec((rblk * K, 16), lambda i: (i, 0)),
        pl.BlockSpec((rblk, c), lambda i: (i, 0)),
        pl.BlockSpec((rblk, c), lambda i: (i, 0)),
        pl.BlockSpec((rblk, c), lambda i: (i, 0)),
        pl.BlockSpec((rblk, 16), lambda i: (i, 0)),
    ] + [full(a) for a in args[7:]]
    kern = functools.partial(_conv_kernel, rblk=rblk, k=K, c=c)
    return pl.pallas_call(
        kern,
        grid=(n // rblk,),
        in_specs=in_specs,
        out_specs=pl.BlockSpec((rblk, c), lambda i: (i, 0)),
        out_shape=jax.ShapeDtypeStruct((n, c), jnp.float32),
    )(*args)


def _lna_kernel(x_ref, w_ref, b_ref, g_ref, bb_ref, o_ref):
    y = _mm(x_ref[...], w_ref[...]) + b_ref[...]
    o_ref[...] = jnp.maximum(_ln(y, g_ref[...], bb_ref[...]), 0.0)


def _lin_norm_act(p, x):
    n, cin = x.shape
    c = p["lin"]["W"].shape[0]
    rblk = min(256, n)
    args = [x, p["lin"]["W"].T, p["lin"]["b"][None, :],
            p["ln"]["g"][None, :], p["ln"]["b"][None, :]]
    full = lambda a: pl.BlockSpec(a.shape, lambda i: (0,) * a.ndim)
    in_specs = [pl.BlockSpec((rblk, cin), lambda i: (i, 0))]
    in_specs += [full(a) for a in args[1:]]
    return pl.pallas_call(
        _lna_kernel,
        grid=(n // rblk,),
        in_specs=in_specs,
        out_specs=pl.BlockSpec((rblk, c), lambda i: (i, 0)),
        out_shape=jax.ShapeDtypeStruct((n, c), jnp.float32),
    )(*args)


def _kmax_kernel(g_ref, o_ref, *, rblk, c):
    o_ref[...] = jnp.max(g_ref[...].reshape(rblk, K, c), axis=1)


def _kmax(g, m, c):
    rblk = min(256, m)
    return pl.pallas_call(
        functools.partial(_kmax_kernel, rblk=rblk, c=c),
        grid=(m // rblk,),
        in_specs=[pl.BlockSpec((rblk * K, c), lambda i: (i, 0))],
        out_specs=pl.BlockSpec((rblk, c), lambda i: (i, 0)),
        out_shape=jax.ShapeDtypeStruct((m, c), jnp.float32),
    )(g)


def _up_kernel(x_ref, d3_ref, g3_ref, w_ref, b_ref, g_ref, bb_ref, o_ref,
               *, rblk, c):
    d3 = d3_ref[...]
    w = 1.0 / jnp.maximum(d3, 1e-16)
    w = w / jnp.sum(w, axis=1, keepdims=True)
    g3 = g3_ref[...].reshape(rblk, 3, c)
    xi = jnp.sum(g3 * w[:, :, None], axis=1)
    y = _mm(x_ref[...], w_ref[...]) + b_ref[...]
    o_ref[...] = jnp.maximum(_ln(y, g_ref[...], bb_ref[...]), 0.0) + xi


def _transition_up(p, x_skip, x_sub, pos16, pos_sub16):
    n = x_skip.shape[0]
    cin = x_skip.shape[1]
    xs = _lin_norm_act(p["mlp_sub"], x_sub)
    idx3, d3 = _knn_topk(pos16, pos_sub16.T, 3, excl_diag=False)
    (g3,) = _sc_gather_many([xs], idx3.reshape(-1))
    rblk = min(256, n)
    args = [x_skip, d3, g3, p["mlp"]["lin"]["W"].T,
            p["mlp"]["lin"]["b"][None, :],
            p["mlp"]["ln"]["g"][None, :], p["mlp"]["ln"]["b"][None, :]]
    full = lambda a: pl.BlockSpec(a.shape, lambda i: (0,) * a.ndim)
    in_specs = [
        pl.BlockSpec((rblk, cin), lambda i: (i, 0)),
        pl.BlockSpec((rblk, 3), lambda i: (i, 0)),
        pl.BlockSpec((rblk * 3, cin), lambda i: (i, 0)),
    ] + [full(a) for a in args[3:]]
    return pl.pallas_call(
        functools.partial(_up_kernel, rblk=rblk, c=cin),
        grid=(n // rblk,),
        in_specs=in_specs,
        out_specs=pl.BlockSpec((rblk, cin), lambda i: (i, 0)),
        out_shape=jax.ShapeDtypeStruct((n, cin), jnp.float32),
    )(*args)


def _pad16(pos):
    return jnp.pad(pos, ((0, 0), (0, 13)))


def kernel(x, pos, batch, params):
    del batch
    n0 = pos.shape[0]
    pos16 = [_pad16(pos)]
    idx = [_knn_topk(pos16[0], pos16[0].T, K, excl_diag=True)[0]]

    out_x = []
    x = _transformer_block(params["t_in"], x, pos16[0], idx[0])
    out_x.append(x)
    n = n0
    for i in range(2):
        sub_pos = _fps(pos16[i][:, :3], RATIO)
        m = sub_pos.shape[0]
        sp16 = _pad16(sub_pos)
        ai, _ = _knn_topk(sp16, pos16[i].T, K, excl_diag=False)
        xt = _lin_norm_act(params["td"][i]["mlp"], x)
        (g,) = _sc_gather_many([xt], ai.reshape(-1))
        c = xt.shape[1]
        x = _kmax(g, m, c)
        pos16.append(sp16)
        idx.append(_knn_topk(sp16, sp16.T, K, excl_diag=True)[0])
        x = _transformer_block(params["tfd"][i], x, sp16, idx[i + 1])
        out_x.append(x)
        n = m

    x = _transformer_block(params["t_sum"], x, pos16[2], idx[2],
                           pre=params["mlp_summit"])

    for i in range(2):
        lev = 1 - i
        x = _transition_up(params["tu"][lev], out_x[lev], x,
                           pos16[lev], pos16[lev + 1])
        x = _transformer_block(params["tfu"][lev], x, pos16[lev], idx[lev])
    return x


# speedup vs baseline: 11.2869x; 1.0278x over previous
import functools

import jax
import jax.numpy as jnp
from jax import lax
from jax.experimental import pallas as pl
from jax.experimental.pallas import tpu as pltpu
from jax.experimental.pallas import tpu_sc as plsc

N_POINTS = 8192
K = 16
RATIO = 0.25
EPS_LN = 1e-5

_SC_CORES = 2
_SC_SUBCORES = 16
_SC_WORKERS = _SC_CORES * _SC_SUBCORES


def _sc_gather_many(tables, idx):
    B = idx.shape[0]
    assert B % (8 * _SC_WORKERS) == 0, B
    b_per_w = B // _SC_WORKERS
    dims = [int(t.shape[1]) for t in tables]
    row_bytes = 4 * sum(dims)
    chunk = b_per_w
    while chunk % 8 == 0 and (chunk // 2) % 8 == 0 and chunk * row_bytes > 262144:
        chunk //= 2
    nch = b_per_w // chunk

    mesh = plsc.VectorSubcoreMesh(core_axis_name="c", subcore_axis_name="s")
    out_type = [jax.ShapeDtypeStruct((B, d), jnp.float32) for d in dims]
    scratch = (
        [pltpu.VMEM((chunk,), jnp.int32)]
        + [pltpu.VMEM((chunk, d), jnp.float32) for d in dims]
        + [pltpu.SemaphoreType.DMA]
    )
    nt = len(tables)

    @functools.partial(
        pl.kernel, out_type=out_type, mesh=mesh, scratch_types=scratch,
        compiler_params=pltpu.CompilerParams(use_tc_tiling_on_sc=False))
    def gather_kernel(*refs):
        tbls = refs[:nt]
        idx_hbm = refs[nt]
        outs = refs[nt + 1:2 * nt + 1]
        idx_v = refs[2 * nt + 1]
        rows = refs[2 * nt + 2:3 * nt + 2]
        sem = refs[3 * nt + 2]
        wid = lax.axis_index("s") * _SC_CORES + lax.axis_index("c")
        base = wid * b_per_w

        def body(gi, carry):
            off = base + gi * chunk
            pltpu.sync_copy(idx_hbm.at[pl.ds(off, chunk)], idx_v)
            copies = [pltpu.async_copy(tbls[t].at[idx_v], rows[t], sem)
                      for t in range(nt)]
            for t in range(nt):
                copies[t].wait()
            for t in range(nt):
                pltpu.sync_copy(rows[t], outs[t].at[pl.ds(off, chunk)])
            return carry

        lax.fori_loop(0, nch, body, 0)

    return gather_kernel(*tables, idx)


def _knn_kernel(a_ref, bt_ref, oi_ref, ov_ref, *, k, nb, rblk, excl_diag):
    pid = pl.program_id(0)
    a = a_ref[...]
    bt = bt_ref[...]
    aa = jnp.sum(a * a, axis=1, keepdims=True)
    bb = jnp.sum(bt * bt, axis=0, keepdims=True)
    ab = _mm(a, bt)
    d = (aa + bb) - 2.0 * ab
    col = lax.broadcasted_iota(jnp.int32, (rblk, nb), 1)
    if excl_diag:
        row = lax.broadcasted_iota(jnp.int32, (rblk, nb), 0) + pid * rblk
        d = jnp.where(col == row, jnp.inf, d)
    for t in range(k):
        mval = jnp.min(d, axis=1, keepdims=True)
        cand = jnp.where(d == mval, col, nb)
        midx = jnp.min(cand, axis=1)
        oi_ref[:, t] = midx
        ov_ref[:, t] = mval[:, 0]
        d = jnp.where(col == midx[:, None], jnp.inf, d)


def _knn_topk(a16, bt16, k, excl_diag):
    na = a16.shape[0]
    nb = bt16.shape[1]
    rblk = min(256, na)
    grid = (na // rblk,)
    kern = functools.partial(_knn_kernel, k=k, nb=nb, rblk=rblk,
                             excl_diag=excl_diag)
    return pl.pallas_call(
        kern,
        grid=grid,
        in_specs=[
            pl.BlockSpec((rblk, 16), lambda i: (i, 0)),
            pl.BlockSpec((16, nb), lambda i: (0, 0)),
        ],
        out_specs=[
            pl.BlockSpec((rblk, k), lambda i: (i, 0)),
            pl.BlockSpec((rblk, k), lambda i: (i, 0)),
        ],
        out_shape=[
            jax.ShapeDtypeStruct((na, k), jnp.int32),
            jax.ShapeDtypeStruct((na, k), jnp.float32),
        ],
    )(a16, bt16)


def _fps_kernel(px_ref, py_ref, pz_ref, o_ref, *, n, m, nrows):
    px = px_ref[...]
    py = py_ref[...]
    pz = pz_ref[...]
    qx0 = px[0, 0]
    qy0 = py[0, 0]
    qz0 = pz[0, 0]
    dx = px - qx0
    dy = py - qy0
    dz = pz - qz0
    d0 = (dx * dx + dy * dy) + dz * dz
    ii = (lax.broadcasted_iota(jnp.int32, (nrows, 128), 0) * 128
          + lax.broadcasted_iota(jnp.int32, (nrows, 128), 1))
    lane = lax.broadcasted_iota(jnp.int32, (1, 128), 1)

    def red2(x, op):
        return op(op(x, axis=0, keepdims=True), axis=1, keepdims=True)

    def qrow(qx, qy, qz):
        return jnp.where(lane == 0, qx,
                         jnp.where(lane == 1, qy,
                                   jnp.where(lane == 2, qz, 0.0)))

    o_ref[0:1, :] = qrow(qx0, qy0, qz0)

    def body(i, d):
        mval = red2(d, jnp.max)
        sel = d == mval
        nxt = red2(jnp.where(sel, ii, n), jnp.min)
        psel = ii == nxt
        qx = red2(jnp.where(psel, px, 0.0), jnp.sum)
        qy = red2(jnp.where(psel, py, 0.0), jnp.sum)
        qz = red2(jnp.where(psel, pz, 0.0), jnp.sum)
        o_ref[pl.ds(i, 1), :] = qrow(qx, qy, qz)
        ex = px - qx
        ey = py - qy
        ez = pz - qz
        dn = (ex * ex + ey * ey) + ez * ez
        return jnp.minimum(d, dn)

    lax.fori_loop(1, m, body, d0)


def _fps(pos, ratio):
    n = pos.shape[0]
    m = int(n * ratio)
    nrows = n // 128
    cols = [pos[:, i].reshape(nrows, 128) for i in range(3)]
    kern = functools.partial(_fps_kernel, n=n, m=m, nrows=nrows)
    wide = pl.pallas_call(
        kern,
        in_specs=[pl.BlockSpec((nrows, 128), lambda: (0, 0))] * 3,
        out_specs=pl.BlockSpec((m, 128), lambda: (0, 0)),
        out_shape=jax.ShapeDtypeStruct((m, 128), jnp.float32),
    )(*cols)
    return wide[:, :3]


def _ln(x, g, b):
    mu = jnp.mean(x, axis=-1, keepdims=True)
    v = jnp.mean((x - mu) ** 2, axis=-1, keepdims=True)
    return (x - mu) / jnp.sqrt(v + EPS_LN) * g + b


def _mm(x, wt):
    return jnp.dot(x.astype(jnp.bfloat16), wt.astype(jnp.bfloat16),
                   preferred_element_type=jnp.float32)


def _pre_kernel(x_ref, wpre_ref, bpre_ref, win_ref, bin_ref, g_ref, b_ref,
                wsrc_ref, wdst_ref, wl_ref, adst_ref, asrc_ref, xl_ref,
                *, has_pre):
    x = x_ref[...]
    if has_pre:
        x = jnp.maximum(_mm(x, wpre_ref[...]) + bpre_ref[...], 0.0)
    x1 = jnp.maximum(_mm(x, win_ref[...]) + bin_ref[...], 0.0)
    xn = _ln(x1, g_ref[...], b_ref[...])
    adst_ref[...] = _mm(xn, wdst_ref[...])
    asrc_ref[...] = _mm(xn, wsrc_ref[...])
    xl_ref[...] = _mm(xn, wl_ref[...])


def _block_prologue(x, p, pre=None):
    n, cin = x.shape
    c = p["lin"]["W"].shape[0]
    rblk = min(256, n)
    if pre is None:
        wpre = jnp.zeros((cin, cin), jnp.float32)
        bpre = jnp.zeros((1, cin), jnp.float32)
    else:
        wpre = pre["W"].T
        bpre = pre["b"][None, :]
    args = [
        x, wpre, bpre,
        p["lin_in"]["W"].T, p["lin_in"]["b"][None, :],
        p["ln"]["g"][None, :], p["ln"]["b"][None, :],
        p["lin_src"]["W"].T, p["lin_dst"]["W"].T, p["lin"]["W"].T,
    ]
    full = lambda a: pl.BlockSpec(a.shape, lambda i: (0,) * a.ndim)
    in_specs = [pl.BlockSpec((rblk, cin), lambda i: (i, 0))]
    in_specs += [full(a) for a in args[1:]]
    kern = functools.partial(_pre_kernel, has_pre=pre is not None)
    return pl.pallas_call(
        kern,
        grid=(n // rblk,),
        in_specs=in_specs,
        out_specs=[pl.BlockSpec((rblk, c), lambda i: (i, 0))] * 3,
        out_shape=[jax.ShapeDtypeStruct((n, c), jnp.float32)] * 3,
    )(*args)


def _conv_kernel(gsrc_ref, gxl_ref, gpos_ref, adst_ref, asrc_ref, xls_ref,
                 pos_ref, p1_ref, p1b_ref, p2_ref, p2b_ref,
                 a1_ref, a1b_ref, a2_ref, a2b_ref, wo_ref, wob_ref,
                 o_ref, *, rblk, k, c):
    gsrc = gsrc_ref[...].reshape(rblk, k, c)
    gxl = gxl_ref[...].reshape(rblk, k, c)
    gpos = gpos_ref[...].reshape(rblk, k, 16)
    adst = adst_ref[...]
    asrc_i = asrc_ref[...]
    xl_i = xls_ref[...]
    pos_i = pos_ref[...]
    p1 = p1_ref[...]
    p1b = p1b_ref[...]
    p2 = p2_ref[...]
    p2b = p2b_ref[...]
    a1 = a1_ref[...]
    a1b = a1b_ref[...]
    a2 = a2_ref[...]
    a2b = a2b_ref[...]

    def mlp(h, w1, b1, w2, b2):
        h = jnp.maximum(_mm(h, w1) + b1, 0.0)
        return jnp.maximum(_mm(h, w2) + b2, 0.0)

    pm = (pos_i[:, None, :] - gpos).reshape(rblk * k, 16)
    delta = mlp(pm, p1, p1b, p2, p2b).reshape(rblk, k, c)
    alpha = adst[:, None, :] - gsrc + delta
    alpha = mlp(alpha.reshape(rblk * k, c), a1, a1b, a2, a2b)
    alpha = alpha.reshape(rblk, k, c)
    hs = jnp.maximum(p1b, 0.0)
    ds = jnp.maximum(_mm(hs, p2) + p2b, 0.0)
    alpha_s = mlp(adst - asrc_i + ds, a1, a1b, a2, a2b)

    amax = jnp.maximum(jnp.max(alpha, axis=1), alpha_s)
    e = jnp.exp(alpha - amax[:, None, :])
    es = jnp.exp(alpha_s - amax)
    denom = jnp.sum(e, axis=1) + es
    num = jnp.sum(e * (gxl + delta), axis=1) + es * (xl_i + ds)
    conv = num / (denom + 1e-16)
    o_ref[...] = jnp.maximum(_mm(conv, wo_ref[...]) + wob_ref[...], 0.0)


def _transformer_block(p, x, pos16, idx, pre=None):
    n = x.shape[0]
    c = p["lin"]["W"].shape[0]
    adst, asrc, xl = _block_prologue(x, p, pre=pre)
    gsrc, gxl, gpos = _sc_gather_many([asrc, xl, pos16], idx.reshape(-1))

    rblk = max(32, min(256, (256 * 64) // c))
    rblk = min(rblk, n)
    p1 = jnp.zeros((16, 64), jnp.float32).at[:3].set(p["pos_nn"]["l1"]["W"].T)
    args = [
        gsrc, gxl, gpos, adst, asrc, xl, pos16,
        p1, p["pos_nn"]["l1"]["b"][None, :],
        p["pos_nn"]["l2"]["W"].T, p["pos_nn"]["l2"]["b"][None, :],
        p["attn_nn"]["l1"]["W"].T, p["attn_nn"]["l1"]["b"][None, :],
        p["attn_nn"]["l2"]["W"].T, p["attn_nn"]["l2"]["b"][None, :],
        p["lin_out"]["W"].T, p["lin_out"]["b"][None, :],
    ]
    full = lambda a: pl.BlockSpec(a.shape, lambda i: (0,) * a.ndim)
    in_specs = [
        pl.BlockSpec((rblk * K, c), lambda i: (i, 0)),
        pl.BlockSpec((rblk * K, c), lambda i: (i, 0)),
        pl.BlockSpec((rblk * K, 16), lambda i: (i, 0)),
        pl.BlockSpec((rblk, c), lambda i: (i, 0)),
        pl.BlockSpec((rblk, c), lambda i: (i, 0)),
        pl.BlockSpec((rblk, c), lambda i: (i, 0)),
        pl.BlockSpec((rblk, 16), lambda i: (i, 0)),
    ] + [full(a) for a in args[7:]]
    kern = functools.partial(_conv_kernel, rblk=rblk, k=K, c=c)
    return pl.pallas_call(
        kern,
        grid=(n // rblk,),
        in_specs=in_specs,
        out_specs=pl.BlockSpec((rblk, c), lambda i: (i, 0)),
        out_shape=jax.ShapeDtypeStruct((n, c), jnp.float32),
    )(*args)


def _lna_kernel(x_ref, w_ref, b_ref, g_ref, bb_ref, o_ref):
    y = _mm(x_ref[...], w_ref[...]) + b_ref[...]
    o_ref[...] = jnp.maximum(_ln(y, g_ref[...], bb_ref[...]), 0.0)


def _lin_norm_act(p, x):
    n, cin = x.shape
    c = p["lin"]["W"].shape[0]
    rblk = min(256, n)
    args = [x, p["lin"]["W"].T, p["lin"]["b"][None, :],
            p["ln"]["g"][None, :], p["ln"]["b"][None, :]]
    full = lambda a: pl.BlockSpec(a.shape, lambda i: (0,) * a.ndim)
    in_specs = [pl.BlockSpec((rblk, cin), lambda i: (i, 0))]
    in_specs += [full(a) for a in args[1:]]
    return pl.pallas_call(
        _lna_kernel,
        grid=(n // rblk,),
        in_specs=in_specs,
        out_specs=pl.BlockSpec((rblk, c), lambda i: (i, 0)),
        out_shape=jax.ShapeDtypeStruct((n, c), jnp.float32),
    )(*args)


def _kmax_kernel(g_ref, o_ref, *, rblk, c):
    o_ref[...] = jnp.max(g_ref[...].reshape(rblk, K, c), axis=1)


def _kmax(g, m, c):
    rblk = min(256, m)
    return pl.pallas_call(
        functools.partial(_kmax_kernel, rblk=rblk, c=c),
        grid=(m // rblk,),
        in_specs=[pl.BlockSpec((rblk * K, c), lambda i: (i, 0))],
        out_specs=pl.BlockSpec((rblk, c), lambda i: (i, 0)),
        out_shape=jax.ShapeDtypeStruct((m, c), jnp.float32),
    )(g)


def _up_kernel(x_ref, d3_ref, g3_ref, w_ref, b_ref, g_ref, bb_ref, o_ref,
               *, rblk, c):
    d3 = d3_ref[...]
    w = 1.0 / jnp.maximum(d3, 1e-16)
    w = w / jnp.sum(w, axis=1, keepdims=True)
    g3 = g3_ref[...].reshape(rblk, 3, c)
    xi = jnp.sum(g3 * w[:, :, None], axis=1)
    y = _mm(x_ref[...], w_ref[...]) + b_ref[...]
    o_ref[...] = jnp.maximum(_ln(y, g_ref[...], bb_ref[...]), 0.0) + xi


def _transition_up(p, x_skip, x_sub, pos16, pos_sub16):
    n = x_skip.shape[0]
    cin = x_skip.shape[1]
    xs = _lin_norm_act(p["mlp_sub"], x_sub)
    idx3, d3 = _knn_topk(pos16, pos_sub16.T, 3, excl_diag=False)
    (g3,) = _sc_gather_many([xs], idx3.reshape(-1))
    rblk = min(256, n)
    args = [x_skip, d3, g3, p["mlp"]["lin"]["W"].T,
            p["mlp"]["lin"]["b"][None, :],
            p["mlp"]["ln"]["g"][None, :], p["mlp"]["ln"]["b"][None, :]]
    full = lambda a: pl.BlockSpec(a.shape, lambda i: (0,) * a.ndim)
    in_specs = [
        pl.BlockSpec((rblk, cin), lambda i: (i, 0)),
        pl.BlockSpec((rblk, 3), lambda i: (i, 0)),
        pl.BlockSpec((rblk * 3, cin), lambda i: (i, 0)),
    ] + [full(a) for a in args[3:]]
    return pl.pallas_call(
        functools.partial(_up_kernel, rblk=rblk, c=cin),
        grid=(n // rblk,),
        in_specs=in_specs,
        out_specs=pl.BlockSpec((rblk, cin), lambda i: (i, 0)),
        out_shape=jax.ShapeDtypeStruct((n, cin), jnp.float32),
    )(*args)


def _pad16(pos):
    return jnp.pad(pos, ((0, 0), (0, 13)))


def kernel(x, pos, batch, params):
    del batch
    n0 = pos.shape[0]
    pos16 = [_pad16(pos)]
    idx = [_knn_topk(pos16[0], pos16[0].T, K, excl_diag=True)[0]]

    out_x = []
    x = _transformer_block(params["t_in"], x, pos16[0], idx[0])
    out_x.append(x)
    n = n0
    for i in range(2):
        sub_pos = _fps(pos16[i][:, :3], RATIO)
        m = sub_pos.shape[0]
        sp16 = _pad16(sub_pos)
        ai, _ = _knn_topk(sp16, pos16[i].T, K, excl_diag=False)
        xt = _lin_norm_act(params["td"][i]["mlp"], x)
        (g,) = _sc_gather_many([xt], ai.reshape(-1))
        c = xt.shape[1]
        x = _kmax(g, m, c)
        pos16.append(sp16)
        idx.append(_knn_topk(sp16, sp16.T, K, excl_diag=True)[0])
        x = _transformer_block(params["tfd"][i], x, sp16, idx[i + 1])
        out_x.append(x)
        n = m

    x = _transformer_block(params["t_sum"], x, pos16[2], idx[2],
                           pre=params["mlp_summit"])

    for i in range(2):
        lev = 1 - i
        x = _transition_up(params["tu"][lev], out_x[lev], x,
                           pos16[lev], pos16[lev + 1])
        x = _transformer_block(params["tfu"][lev], x, pos16[lev], idx[lev])
    return x
